# baseline (speedup 1.0000x reference)
import functools

import jax
import jax.numpy as jnp
from jax import lax
from jax.experimental import pallas as pl
from jax.experimental.pallas import tpu as pltpu
from jax.experimental.pallas import tpu_sc as plsc

V = 1000000
D = 64
H = 512
O = 128
B = 4096
L = 200

NC = 2
NS = 16
NW = NC * NS
SEQ_PER_W = B // NW
CHUNK = 100
NCHUNK = L // CHUNK
NBUF = 3
INV_L = 1.0 / L


def _pool_body(wi_hbm, table_hbm, out_hbm, idx_v, rows_v, out_v, sem):
    wid = lax.axis_index("s") * NC + lax.axis_index("c")
    base = wid * SEQ_PER_W

    @pl.loop(0, SEQ_PER_W)
    def _(s):
        pltpu.sync_copy(wi_hbm.at[pl.ds((base + s) * NCHUNK, NCHUNK)], idx_v)
        cps = [
            pltpu.async_copy(
                table_hbm.at[idx_v.at[j]],
                rows_v.at[pl.ds(j * CHUNK, CHUNK)],
                sem,
            )
            for j in range(NCHUNK)
        ]
        for cp in cps:
            cp.wait()

        def acc_body(r, accs):
            return tuple(
                a + rows_v[r, pl.ds(c * 16, 16)]
                for c, a in enumerate(accs)
            )

        accs = lax.fori_loop(
            0, L, acc_body,
            tuple(jnp.zeros((16,), jnp.float32) for _ in range(4)),
            unroll=8,
        )
        for c in range(4):
            out_v[s, pl.ds(c * 16, 16)] = accs[c] * INV_L

    pltpu.sync_copy(out_v, out_hbm.at[pl.ds(base, SEQ_PER_W)])


@jax.jit
def _pool(word_indices, table):
    wi = word_indices.reshape(B * NCHUNK, CHUNK)
    mesh = plsc.VectorSubcoreMesh(core_axis_name="c", subcore_axis_name="s")
    return pl.kernel(
        _pool_body,
        out_type=jax.ShapeDtypeStruct((B, D), jnp.float32),
        mesh=mesh,
        scratch_types=[
            pltpu.VMEM((NCHUNK, CHUNK), jnp.int32),
            pltpu.VMEM((L, D), jnp.float32),
            pltpu.VMEM((SEQ_PER_W, D), jnp.float32),
            pltpu.SemaphoreType.DMA,
        ],
        compiler_params=pltpu.CompilerParams(use_tc_tiling_on_sc=False),
    )(wi, table)


BM = 512


def _mlp_body(x_ref, w1t_ref, b1_ref, w2t_ref, b2_ref, out_ref):
    x = x_ref[...]
    h = jnp.dot(x, w1t_ref[...], preferred_element_type=jnp.float32)
    h = jnp.maximum(h + b1_ref[...], 0.0)
    logits = jnp.dot(h, w2t_ref[...], preferred_element_type=jnp.float32)
    logits = logits + b2_ref[...]
    m = jnp.max(logits, axis=1, keepdims=True)
    lse = jnp.log(jnp.sum(jnp.exp(logits - m), axis=1, keepdims=True)) + m
    out_ref[...] = logits - lse


@jax.jit
def _mlp(x, W1, b1, W2, b2):
    w1t = W1.T
    w2t = W2.T
    b1r = b1.reshape(1, H)
    b2r = b2.reshape(1, O)
    grid = (B // BM,)
    return pl.pallas_call(
        _mlp_body,
        grid=grid,
        in_specs=[
            pl.BlockSpec((BM, D), lambda i: (i, 0)),
            pl.BlockSpec((D, H), lambda i: (0, 0)),
            pl.BlockSpec((1, H), lambda i: (0, 0)),
            pl.BlockSpec((H, O), lambda i: (0, 0)),
            pl.BlockSpec((1, O), lambda i: (0, 0)),
        ],
        out_specs=pl.BlockSpec((BM, O), lambda i: (i, 0)),
        out_shape=jax.ShapeDtypeStruct((B, O), jnp.float32),
    )(x, w1t, b1r, w2t, b2r)


def kernel(word_indices, table, W1, b1, W2, b2):
    pooled = _pool(word_indices, table)
    return _mlp(pooled, W1, b1, W2, b2)


# speedup vs baseline: 1.2115x; 1.2115x over previous
import functools

import jax
import jax.numpy as jnp
from jax import lax
from jax.experimental import pallas as pl
from jax.experimental.pallas import tpu as pltpu
from jax.experimental.pallas import tpu_sc as plsc

V = 1000000
D = 64
H = 512
O = 128
B = 4096
L = 200

NC = 2
NS = 16
NW = NC * NS
SEQ_PER_W = B // NW
CHUNK = 100
NCHUNK = L // CHUNK
NBUF = 3
INV_L = 1.0 / L


KSEQ = 4
NBATCH = SEQ_PER_W // KSEQ


def _pool_body(wi_hbm, table_hbm, out_hbm, idx_v, rows_v, out_v, sems):
    wid = lax.axis_index("s") * NC + lax.axis_index("c")
    base = wid * SEQ_PER_W

    def start_batch(t, b):
        pltpu.sync_copy(
            wi_hbm.at[pl.ds((base + t * KSEQ) * NCHUNK, KSEQ * NCHUNK)],
            idx_v.at[b],
        )
        for k in range(KSEQ):
            for j in range(NCHUNK):
                pltpu.async_copy(
                    table_hbm.at[idx_v.at[b, k * NCHUNK + j]],
                    rows_v.at[b, k, pl.ds(j * CHUNK, CHUNK)],
                    sems.at[b],
                )

    def wait_batch(b):
        for k in range(KSEQ):
            for j in range(NCHUNK):
                pltpu.make_async_copy(
                    table_hbm.at[idx_v.at[b, k * NCHUNK + j]],
                    rows_v.at[b, k, pl.ds(j * CHUNK, CHUNK)],
                    sems.at[b],
                ).wait()

    def accum_batch(t, b):
        for k in range(KSEQ):
            def acc_body(r, accs):
                return tuple(
                    a + rows_v[b, k, r, pl.ds(c * 16, 16)]
                    for c, a in enumerate(accs)
                )

            accs = lax.fori_loop(
                0, L, acc_body,
                tuple(jnp.zeros((16,), jnp.float32) for _ in range(4)),
                unroll=8,
            )
            for c in range(4):
                out_v[t * KSEQ + k, pl.ds(c * 16, 16)] = accs[c] * INV_L

    start_batch(0, 0)

    @pl.loop(0, NBATCH - 2, step=2)
    def _(t0):
        for b in range(2):
            t = t0 + b
            wait_batch(b)
            start_batch(t + 1, 1 - b)
            accum_batch(t, b)

    wait_batch(0)
    start_batch(NBATCH - 1, 1)
    accum_batch(NBATCH - 2, 0)
    wait_batch(1)
    accum_batch(NBATCH - 1, 1)

    pltpu.sync_copy(out_v, out_hbm.at[pl.ds(base, SEQ_PER_W)])


@jax.jit
def _pool(word_indices, table):
    wi = word_indices.reshape(B * NCHUNK, CHUNK)
    mesh = plsc.VectorSubcoreMesh(core_axis_name="c", subcore_axis_name="s")
    return pl.kernel(
        _pool_body,
        out_type=jax.ShapeDtypeStruct((B, D), jnp.float32),
        mesh=mesh,
        scratch_types=[
            pltpu.VMEM((2, KSEQ * NCHUNK, CHUNK), jnp.int32),
            pltpu.VMEM((2, KSEQ, L, D), jnp.float32),
            pltpu.VMEM((SEQ_PER_W, D), jnp.float32),
            pltpu.SemaphoreType.DMA((2,)),
        ],
        compiler_params=pltpu.CompilerParams(use_tc_tiling_on_sc=False),
    )(wi, table)


BM = 512


def _mlp_body(x_ref, w1t_ref, b1_ref, w2t_ref, b2_ref, out_ref):
    x = x_ref[...]
    h = jnp.dot(x, w1t_ref[...], preferred_element_type=jnp.float32)
    h = jnp.maximum(h + b1_ref[...], 0.0)
    logits = jnp.dot(h, w2t_ref[...], preferred_element_type=jnp.float32)
    logits = logits + b2_ref[...]
    m = jnp.max(logits, axis=1, keepdims=True)
    lse = jnp.log(jnp.sum(jnp.exp(logits - m), axis=1, keepdims=True)) + m
    out_ref[...] = logits - lse


@jax.jit
def _mlp(x, W1, b1, W2, b2):
    w1t = W1.T
    w2t = W2.T
    b1r = b1.reshape(1, H)
    b2r = b2.reshape(1, O)
    grid = (B // BM,)
    return pl.pallas_call(
        _mlp_body,
        grid=grid,
        in_specs=[
            pl.BlockSpec((BM, D), lambda i: (i, 0)),
            pl.BlockSpec((D, H), lambda i: (0, 0)),
            pl.BlockSpec((1, H), lambda i: (0, 0)),
            pl.BlockSpec((H, O), lambda i: (0, 0)),
            pl.BlockSpec((1, O), lambda i: (0, 0)),
        ],
        out_specs=pl.BlockSpec((BM, O), lambda i: (i, 0)),
        out_shape=jax.ShapeDtypeStruct((B, O), jnp.float32),
    )(x, w1t, b1r, w2t, b2r)


def kernel(word_indices, table, W1, b1, W2, b2):
    pooled = _pool(word_indices, table)
    return _mlp(pooled, W1, b1, W2, b2)
